# baseline (speedup 1.0000x reference)
import functools

import jax
import jax.numpy as jnp
import numpy as np
from jax import lax
from jax.experimental import pallas as pl
from jax.experimental.pallas import tpu as pltpu
from jax.experimental.pallas import tpu_sc as plsc

DIM = 128
NUM_ACTION = 100000
BATCH = 1024
CB = 1024
NBLK = (NUM_ACTION + CB - 1) // CB

TC_BLKS = 73
SC_COL0 = TC_BLKS * CB
SC_COLS = NUM_ACTION - SC_COL0
SC_NW = 32
SC_ROWS = BATCH // SC_NW
SC_NB = 3
SC_BC = SC_COLS // SC_NB
SC_ITERS = SC_BC // 32

_TINY = np.float32(np.finfo(np.float32).tiny)
_KS1 = np.uint32(42)
_KS2 = np.uint32(0x1BD11BDA) ^ np.uint32(42)
_ROT_A = (13, 15, 26, 6)
_ROT_B = (17, 29, 16, 24)
_LN2 = np.float32(0.6931471805599453)
_SQRT2 = np.float32(1.4142135623730951)
_IMAX = np.int32(2**31 - 1)


def _rotl(x, d):
    return (x << np.uint32(d)) | (x >> np.uint32(32 - d))


def _rounds(x0, x1, rots):
    for r in rots:
        x0 = x0 + x1
        x1 = _rotl(x1, r)
        x1 = x0 ^ x1
    return x0, x1


def _uniform_bits(flat_idx_u32):
    x0 = jnp.zeros_like(flat_idx_u32)
    x1 = flat_idx_u32 + _KS1
    x0, x1 = _rounds(x0, x1, _ROT_A)
    x0 = x0 + _KS1
    x1 = x1 + (_KS2 + np.uint32(1))
    x0, x1 = _rounds(x0, x1, _ROT_B)
    x0 = x0 + _KS2
    x1 = x1 + np.uint32(2)
    x0, x1 = _rounds(x0, x1, _ROT_A)
    x1 = x1 + (_KS1 + np.uint32(3))
    x0, x1 = _rounds(x0, x1, _ROT_B)
    x0 = x0 + _KS1
    x1 = x1 + (_KS2 + np.uint32(4))
    x0, x1 = _rounds(x0, x1, _ROT_A)
    x0 = x0 + _KS2
    x1 = x1 + np.uint32(5)
    bits = x0 ^ x1
    float_bits = (bits >> np.uint32(9)) | np.uint32(0x3F800000)
    floats = jax.lax.bitcast_convert_type(float_bits, jnp.float32) - np.float32(1.0)
    return jnp.maximum(floats, _TINY)


def _gumbel(flat_idx_u32):
    return -jnp.log(-jnp.log(_uniform_bits(flat_idx_u32)))


def _tc_body(feature_ref, w1t_ref, b1_ref, w2_ref, b2_ref,
             logits_ref, draw_ref, drawv_ref, h_sc, bestv_sc, besti_sc):
    j = pl.program_id(0)

    @pl.when(j == 0)
    def _init():
        h = jnp.dot(feature_ref[...], w1t_ref[...],
                    preferred_element_type=jnp.float32)
        h_sc[...] = jnp.maximum(h + b1_ref[...], 0.0)
        bestv_sc[...] = jnp.full((BATCH, 1), -jnp.inf, jnp.float32)
        besti_sc[...] = jnp.zeros((BATCH, 1), jnp.int32)

    h = h_sc[...]
    logits = jax.lax.dot_general(h, w2_ref[...], (((1,), (1,)), ((), ())),
                                 preferred_element_type=jnp.float32)
    logits = logits + b2_ref[...]
    logits_ref[...] = logits

    @pl.when(j < TC_BLKS)
    def _rng():
        col = j * CB + jax.lax.broadcasted_iota(jnp.int32, (BATCH, CB), 1)
        row = jax.lax.broadcasted_iota(jnp.int32, (BATCH, CB), 0)
        flat = (row * NUM_ACTION + col).astype(jnp.uint32)
        v = logits + _gumbel(flat)
        m = jnp.max(v, axis=1, keepdims=True)
        idx = jnp.min(jnp.where(v == m, col, _IMAX), axis=1, keepdims=True)
        better = m > bestv_sc[...]
        bestv_sc[...] = jnp.where(better, m, bestv_sc[...])
        besti_sc[...] = jnp.where(better, idx, besti_sc[...])

    @pl.when(j == NBLK - 1)
    def _fin():
        draw_ref[...] = besti_sc[...]
        drawv_ref[...] = bestv_sc[...]


def _log_poly(x):
    bits = jax.lax.bitcast_convert_type(x, jnp.int32)
    e = ((bits + np.int32(0x4AFB0D)) >> np.int32(23)) - np.int32(127)
    m = jax.lax.bitcast_convert_type(bits - (e << np.int32(23)), jnp.float32)
    z = m - np.float32(1.0)
    w = z / (z + np.float32(2.0))
    w2 = w * w
    p = w2 * np.float32(2.0 / 9.0) + np.float32(2.0 / 7.0)
    p = p * w2 + np.float32(2.0 / 5.0)
    p = p * w2 + np.float32(2.0 / 3.0)
    p = p * w2 + np.float32(2.0)
    return e.astype(jnp.float32) * _LN2 + w * p


def _gumbel_sc(flat_idx_u32):
    u = _uniform_bits(flat_idx_u32)
    t = -_log_poly(u)
    return -_log_poly(t)


def _sc_kernel_body(featf, w1f, b1p, w2tf, b2f, outv, outi,
                    feat_v, w1_v, b1_v, w2_v, b2_v, h_v, bv_v, bi_v):
    wid = lax.axis_index("s") * np.int32(2) + lax.axis_index("c")
    r0 = wid * SC_ROWS
    pltpu.sync_copy(featf.at[pl.ds(r0 * DIM, SC_ROWS * DIM)], feat_v)
    pltpu.sync_copy(w1f, w1_v)
    pltpu.sync_copy(b1p, b1_v)

    iota16 = lax.iota(jnp.int32, 16)
    neg_inf = jnp.full((16,), -jnp.inf, jnp.float32)
    zeros_i = jnp.zeros((16,), jnp.int32)
    b1vec = b1_v[...]

    def compute_h(r, carry):
        for k in range(8):
            acc = jnp.zeros((16,), jnp.float32)
            for dd in range(DIM // 16):
                f = feat_v[pl.ds(r * DIM + dd * 16, 16)]
                w = w1_v[pl.ds(k * DIM + dd * 16, 16)]
                acc = acc + f * w
            for sh in (8, 4, 2, 1):
                acc = acc + acc.at[iota16 ^ sh].get(mode="promise_in_bounds")
            hk = jnp.maximum(acc + b1vec[k], np.float32(0.0))
            h_v[pl.ds((r * 8 + k) * 16, 16)] = hk
        bv_v[pl.ds(r * 16, 16)] = neg_inf
        bi_v[pl.ds(r * 16, 16)] = zeros_i
        return carry

    lax.fori_loop(0, SC_ROWS, compute_h, 0)

    def block_loop(b, carry):
        for k in range(8):
            pltpu.sync_copy(
                w2tf.at[pl.ds(k * SC_COLS + b * SC_BC, SC_BC)],
                w2_v.at[pl.ds(k * SC_BC, SC_BC)])
        pltpu.sync_copy(b2f.at[pl.ds(b * SC_BC, SC_BC)], b2_v)
        colbase = SC_COL0 + b * SC_BC

        def rowpair_loop(rp, carry2):
            rA = rp * np.int32(2)
            rB = rA + np.int32(1)
            hA = [h_v[pl.ds((rA * 8 + k) * 16, 16)] for k in range(8)]
            hB = [h_v[pl.ds((rB * 8 + k) * 16, 16)] for k in range(8)]
            rowoffA = (r0 + rA) * NUM_ACTION
            rowoffB = (r0 + rB) * NUM_ACTION

            def chunk_loop(i, st):
                bvA, biA, bvB, biB = st
                for cc in range(2):
                    c16 = (i * 2 + cc) * 16
                    w2c = [w2_v[pl.ds(k * SC_BC + c16, 16)] for k in range(8)]
                    b2c = b2_v[pl.ds(c16, 16)]
                    colv = colbase + c16 + iota16
                    for which in (0, 1):
                        hh = hA if which == 0 else hB
                        rowoff = rowoffA if which == 0 else rowoffB
                        l = b2c
                        for k in range(8):
                            l = l + hh[k] * w2c[k]
                        flat = (rowoff + colv).astype(jnp.uint32)
                        v = l + _gumbel_sc(flat)
                        if which == 0:
                            better = v > bvA
                            bvA = jnp.where(better, v, bvA)
                            biA = jnp.where(better, colv, biA)
                        else:
                            better = v > bvB
                            bvB = jnp.where(better, v, bvB)
                            biB = jnp.where(better, colv, biB)
                return bvA, biA, bvB, biB

            st0 = (bv_v[pl.ds(rA * 16, 16)], bi_v[pl.ds(rA * 16, 16)],
                   bv_v[pl.ds(rB * 16, 16)], bi_v[pl.ds(rB * 16, 16)])
            bvA, biA, bvB, biB = lax.fori_loop(0, SC_ITERS, chunk_loop, st0)
            bv_v[pl.ds(rA * 16, 16)] = bvA
            bi_v[pl.ds(rA * 16, 16)] = biA
            bv_v[pl.ds(rB * 16, 16)] = bvB
            bi_v[pl.ds(rB * 16, 16)] = biB
            return carry2

        lax.fori_loop(0, SC_ROWS // 2, rowpair_loop, 0)
        return carry

    lax.fori_loop(0, SC_NB, block_loop, 0)

    pltpu.sync_copy(bv_v, outv.at[pl.ds(r0 * 16, SC_ROWS * 16)])
    pltpu.sync_copy(bi_v, outi.at[pl.ds(r0 * 16, SC_ROWS * 16)])


def _make_sc_call():
    return functools.partial(
        pl.kernel,
        mesh=plsc.VectorSubcoreMesh(core_axis_name="c", subcore_axis_name="s"),
        out_type=[jax.ShapeDtypeStruct((BATCH * 16,), jnp.float32),
                  jax.ShapeDtypeStruct((BATCH * 16,), jnp.int32)],
        scratch_types=[
            pltpu.VMEM((SC_ROWS * DIM,), jnp.float32),
            pltpu.VMEM((8 * DIM,), jnp.float32),
            pltpu.VMEM((16,), jnp.float32),
            pltpu.VMEM((8 * SC_BC,), jnp.float32),
            pltpu.VMEM((SC_BC,), jnp.float32),
            pltpu.VMEM((SC_ROWS * 8 * 16,), jnp.float32),
            pltpu.VMEM((SC_ROWS * 16,), jnp.float32),
            pltpu.VMEM((SC_ROWS * 16,), jnp.int32),
        ],
    )(_sc_kernel_body)


@jax.jit
def kernel(feature, W1, b1, W2, b2):
    w1t = W1.T
    b1r = b1.reshape(1, 8)
    b2r = b2.reshape(1, NUM_ACTION)
    logits, tc_i, tc_v = pl.pallas_call(
        _tc_body,
        grid=(NBLK,),
        in_specs=[
            pl.BlockSpec((BATCH, DIM), lambda j: (0, 0)),
            pl.BlockSpec((DIM, 8), lambda j: (0, 0)),
            pl.BlockSpec((1, 8), lambda j: (0, 0)),
            pl.BlockSpec((CB, 8), lambda j: (j, 0)),
            pl.BlockSpec((1, CB), lambda j: (0, j)),
        ],
        out_specs=[
            pl.BlockSpec((BATCH, CB), lambda j: (0, j)),
            pl.BlockSpec((BATCH, 1), lambda j: (0, 0)),
            pl.BlockSpec((BATCH, 1), lambda j: (0, 0)),
        ],
        out_shape=[
            jax.ShapeDtypeStruct((BATCH, NUM_ACTION), jnp.float32),
            jax.ShapeDtypeStruct((BATCH, 1), jnp.int32),
            jax.ShapeDtypeStruct((BATCH, 1), jnp.float32),
        ],
        scratch_shapes=[
            pltpu.VMEM((BATCH, 8), jnp.float32),
            pltpu.VMEM((BATCH, 1), jnp.float32),
            pltpu.VMEM((BATCH, 1), jnp.int32),
        ],
    )(feature, w1t, b1r, W2, b2r)

    w2t_sc = W2.T[:, SC_COL0:]
    sc_v, sc_i = _make_sc_call()(feature.reshape(-1), W1.reshape(-1),
                                 jnp.pad(b1, (0, 8)), w2t_sc.reshape(-1),
                                 b2[SC_COL0:])

    V = jnp.concatenate([tc_v, sc_v.reshape(BATCH, 16)], axis=1)
    I = jnp.concatenate([tc_i, sc_i.reshape(BATCH, 16)], axis=1)
    m = jnp.max(V, axis=1, keepdims=True)
    draw = jnp.min(jnp.where(V == m, I, _IMAX), axis=1, keepdims=True)
    return (logits, draw)


# speedup vs baseline: 1.0352x; 1.0352x over previous
import functools

import jax
import jax.numpy as jnp
import numpy as np
from jax import lax
from jax.experimental import pallas as pl
from jax.experimental.pallas import tpu as pltpu
from jax.experimental.pallas import tpu_sc as plsc

DIM = 128
NUM_ACTION = 100000
BATCH = 1024
CB = 1024
NBLK = (NUM_ACTION + CB - 1) // CB

TC_BLKS = 75
SC_COL0 = TC_BLKS * CB
SC_COLS = NUM_ACTION - SC_COL0
SC_NW = 32
SC_ROWS = BATCH // SC_NW
SC_NB = 5
SC_BC = SC_COLS // SC_NB
SC_ITERS = SC_BC // 32

_TINY = np.float32(np.finfo(np.float32).tiny)
_KS1 = np.uint32(42)
_KS2 = np.uint32(0x1BD11BDA) ^ np.uint32(42)
_ROT_A = (13, 15, 26, 6)
_ROT_B = (17, 29, 16, 24)
_LN2 = np.float32(0.6931471805599453)
_SQRT2 = np.float32(1.4142135623730951)
_IMAX = np.int32(2**31 - 1)


def _rotl(x, d):
    return (x << np.uint32(d)) | (x >> np.uint32(32 - d))


def _rounds(x0, x1, rots):
    for r in rots:
        x0 = x0 + x1
        x1 = _rotl(x1, r)
        x1 = x0 ^ x1
    return x0, x1


def _uniform_bits(flat_idx_u32):
    x0 = jnp.zeros_like(flat_idx_u32)
    x1 = flat_idx_u32 + _KS1
    x0, x1 = _rounds(x0, x1, _ROT_A)
    x0 = x0 + _KS1
    x1 = x1 + (_KS2 + np.uint32(1))
    x0, x1 = _rounds(x0, x1, _ROT_B)
    x0 = x0 + _KS2
    x1 = x1 + np.uint32(2)
    x0, x1 = _rounds(x0, x1, _ROT_A)
    x1 = x1 + (_KS1 + np.uint32(3))
    x0, x1 = _rounds(x0, x1, _ROT_B)
    x0 = x0 + _KS1
    x1 = x1 + (_KS2 + np.uint32(4))
    x0, x1 = _rounds(x0, x1, _ROT_A)
    x0 = x0 + _KS2
    x1 = x1 + np.uint32(5)
    bits = x0 ^ x1
    float_bits = (bits >> np.uint32(9)) | np.uint32(0x3F800000)
    floats = jax.lax.bitcast_convert_type(float_bits, jnp.float32) - np.float32(1.0)
    return jnp.maximum(floats, _TINY)


def _gumbel(flat_idx_u32):
    return -jnp.log(-jnp.log(_uniform_bits(flat_idx_u32)))


def _tc_body(feature_ref, w1t_ref, b1_ref, w2_ref, b2_ref,
             logits_ref, draw_ref, drawv_ref, h_sc, bestv_sc, besti_sc):
    j = pl.program_id(0)

    @pl.when(j == 0)
    def _init():
        h = jnp.dot(feature_ref[...], w1t_ref[...],
                    preferred_element_type=jnp.float32)
        h_sc[...] = jnp.maximum(h + b1_ref[...], 0.0)
        bestv_sc[...] = jnp.full((BATCH, 1), -jnp.inf, jnp.float32)
        besti_sc[...] = jnp.zeros((BATCH, 1), jnp.int32)

    h = h_sc[...]
    logits = jax.lax.dot_general(h, w2_ref[...], (((1,), (1,)), ((), ())),
                                 preferred_element_type=jnp.float32)
    logits = logits + b2_ref[...]
    logits_ref[...] = logits

    @pl.when(j < TC_BLKS)
    def _rng():
        col = j * CB + jax.lax.broadcasted_iota(jnp.int32, (BATCH, CB), 1)
        row = jax.lax.broadcasted_iota(jnp.int32, (BATCH, CB), 0)
        flat = (row * NUM_ACTION + col).astype(jnp.uint32)
        v = logits + _gumbel(flat)
        m = jnp.max(v, axis=1, keepdims=True)
        idx = jnp.min(jnp.where(v == m, col, _IMAX), axis=1, keepdims=True)
        better = m > bestv_sc[...]
        bestv_sc[...] = jnp.where(better, m, bestv_sc[...])
        besti_sc[...] = jnp.where(better, idx, besti_sc[...])

    @pl.when(j == NBLK - 1)
    def _fin():
        draw_ref[...] = besti_sc[...]
        drawv_ref[...] = bestv_sc[...]


def _log_poly(x):
    bits = jax.lax.bitcast_convert_type(x, jnp.int32)
    e = ((bits + np.int32(0x4AFB0D)) >> np.int32(23)) - np.int32(127)
    m = jax.lax.bitcast_convert_type(bits - (e << np.int32(23)), jnp.float32)
    z = m - np.float32(1.0)
    w = z / (z + np.float32(2.0))
    w2 = w * w
    p = w2 * np.float32(2.0 / 9.0) + np.float32(2.0 / 7.0)
    p = p * w2 + np.float32(2.0 / 5.0)
    p = p * w2 + np.float32(2.0 / 3.0)
    p = p * w2 + np.float32(2.0)
    return e.astype(jnp.float32) * _LN2 + w * p


def _gumbel_sc(flat_idx_u32):
    u = _uniform_bits(flat_idx_u32)
    t = -_log_poly(u)
    return -_log_poly(t)


def _sc_kernel_body(featf, w1f, b1p, w2tf, b2f, outv, outi,
                    feat_v, w1_v, b1_v, w2_v, b2_v, h_v, bv_v, bi_v):
    wid = lax.axis_index("s") * np.int32(2) + lax.axis_index("c")
    r0 = wid * SC_ROWS
    pltpu.sync_copy(featf.at[pl.ds(r0 * DIM, SC_ROWS * DIM)], feat_v)
    pltpu.sync_copy(w1f, w1_v)
    pltpu.sync_copy(b1p, b1_v)

    iota16 = lax.iota(jnp.int32, 16)
    neg_inf = jnp.full((16,), -jnp.inf, jnp.float32)
    zeros_i = jnp.zeros((16,), jnp.int32)
    b1vec = b1_v[...]

    def compute_h(r, carry):
        for k in range(8):
            acc = jnp.zeros((16,), jnp.float32)
            for dd in range(DIM // 16):
                f = feat_v[pl.ds(r * DIM + dd * 16, 16)]
                w = w1_v[pl.ds(k * DIM + dd * 16, 16)]
                acc = acc + f * w
            for sh in (8, 4, 2, 1):
                acc = acc + acc.at[iota16 ^ sh].get(mode="promise_in_bounds")
            hk = jnp.maximum(acc + b1vec[k], np.float32(0.0))
            h_v[pl.ds((r * 8 + k) * 16, 16)] = hk
        bv_v[pl.ds(r * 16, 16)] = neg_inf
        bi_v[pl.ds(r * 16, 16)] = zeros_i
        return carry

    lax.fori_loop(0, SC_ROWS, compute_h, 0)

    def block_loop(b, carry):
        for k in range(8):
            pltpu.sync_copy(
                w2tf.at[pl.ds(k * SC_COLS + b * SC_BC, SC_BC)],
                w2_v.at[pl.ds(k * SC_BC, SC_BC)])
        pltpu.sync_copy(b2f.at[pl.ds(b * SC_BC, SC_BC)], b2_v)
        colbase = SC_COL0 + b * SC_BC

        def rowpair_loop(rp, carry2):
            rA = rp * np.int32(2)
            rB = rA + np.int32(1)
            hA = [h_v[pl.ds((rA * 8 + k) * 16, 16)] for k in range(8)]
            hB = [h_v[pl.ds((rB * 8 + k) * 16, 16)] for k in range(8)]
            rowoffA = (r0 + rA) * NUM_ACTION
            rowoffB = (r0 + rB) * NUM_ACTION

            def chunk_loop(i, st):
                bvA, biA, bvB, biB = st
                for cc in range(2):
                    c16 = (i * 2 + cc) * 16
                    w2c = [w2_v[pl.ds(k * SC_BC + c16, 16)] for k in range(8)]
                    b2c = b2_v[pl.ds(c16, 16)]
                    colv = colbase + c16 + iota16
                    for which in (0, 1):
                        hh = hA if which == 0 else hB
                        rowoff = rowoffA if which == 0 else rowoffB
                        l = b2c
                        for k in range(8):
                            l = l + hh[k] * w2c[k]
                        flat = (rowoff + colv).astype(jnp.uint32)
                        v = l + _gumbel_sc(flat)
                        if which == 0:
                            better = v > bvA
                            bvA = jnp.where(better, v, bvA)
                            biA = jnp.where(better, colv, biA)
                        else:
                            better = v > bvB
                            bvB = jnp.where(better, v, bvB)
                            biB = jnp.where(better, colv, biB)
                return bvA, biA, bvB, biB

            st0 = (bv_v[pl.ds(rA * 16, 16)], bi_v[pl.ds(rA * 16, 16)],
                   bv_v[pl.ds(rB * 16, 16)], bi_v[pl.ds(rB * 16, 16)])
            bvA, biA, bvB, biB = lax.fori_loop(0, SC_ITERS, chunk_loop, st0)
            bv_v[pl.ds(rA * 16, 16)] = bvA
            bi_v[pl.ds(rA * 16, 16)] = biA
            bv_v[pl.ds(rB * 16, 16)] = bvB
            bi_v[pl.ds(rB * 16, 16)] = biB
            return carry2

        lax.fori_loop(0, SC_ROWS // 2, rowpair_loop, 0)
        return carry

    lax.fori_loop(0, SC_NB, block_loop, 0)

    pltpu.sync_copy(bv_v, outv.at[pl.ds(r0 * 16, SC_ROWS * 16)])
    pltpu.sync_copy(bi_v, outi.at[pl.ds(r0 * 16, SC_ROWS * 16)])


def _make_sc_call():
    return functools.partial(
        pl.kernel,
        mesh=plsc.VectorSubcoreMesh(core_axis_name="c", subcore_axis_name="s"),
        out_type=[jax.ShapeDtypeStruct((BATCH * 16,), jnp.float32),
                  jax.ShapeDtypeStruct((BATCH * 16,), jnp.int32)],
        scratch_types=[
            pltpu.VMEM((SC_ROWS * DIM,), jnp.float32),
            pltpu.VMEM((8 * DIM,), jnp.float32),
            pltpu.VMEM((16,), jnp.float32),
            pltpu.VMEM((8 * SC_BC,), jnp.float32),
            pltpu.VMEM((SC_BC,), jnp.float32),
            pltpu.VMEM((SC_ROWS * 8 * 16,), jnp.float32),
            pltpu.VMEM((SC_ROWS * 16,), jnp.float32),
            pltpu.VMEM((SC_ROWS * 16,), jnp.int32),
        ],
    )(_sc_kernel_body)


@jax.jit
def kernel(feature, W1, b1, W2, b2):
    w1t = W1.T
    b1r = b1.reshape(1, 8)
    b2r = b2.reshape(1, NUM_ACTION)
    logits, tc_i, tc_v = pl.pallas_call(
        _tc_body,
        grid=(NBLK,),
        in_specs=[
            pl.BlockSpec((BATCH, DIM), lambda j: (0, 0)),
            pl.BlockSpec((DIM, 8), lambda j: (0, 0)),
            pl.BlockSpec((1, 8), lambda j: (0, 0)),
            pl.BlockSpec((CB, 8), lambda j: (j, 0)),
            pl.BlockSpec((1, CB), lambda j: (0, j)),
        ],
        out_specs=[
            pl.BlockSpec((BATCH, CB), lambda j: (0, j)),
            pl.BlockSpec((BATCH, 1), lambda j: (0, 0)),
            pl.BlockSpec((BATCH, 1), lambda j: (0, 0)),
        ],
        out_shape=[
            jax.ShapeDtypeStruct((BATCH, NUM_ACTION), jnp.float32),
            jax.ShapeDtypeStruct((BATCH, 1), jnp.int32),
            jax.ShapeDtypeStruct((BATCH, 1), jnp.float32),
        ],
        scratch_shapes=[
            pltpu.VMEM((BATCH, 8), jnp.float32),
            pltpu.VMEM((BATCH, 1), jnp.float32),
            pltpu.VMEM((BATCH, 1), jnp.int32),
        ],
    )(feature, w1t, b1r, W2, b2r)

    w2t_sc = W2.T[:, SC_COL0:]
    sc_v, sc_i = _make_sc_call()(feature.reshape(-1), W1.reshape(-1),
                                 jnp.pad(b1, (0, 8)), w2t_sc.reshape(-1),
                                 b2[SC_COL0:])

    del tc_v
    cand = jnp.concatenate([tc_i, sc_i.reshape(BATCH, 16)], axis=1)
    l_cand = jnp.take_along_axis(logits, cand, axis=1)
    rows = jnp.arange(BATCH, dtype=jnp.int32)[:, None]
    flat = (rows * NUM_ACTION + cand).astype(jnp.uint32)
    v = l_cand + _gumbel(flat)
    m = jnp.max(v, axis=1, keepdims=True)
    draw = jnp.min(jnp.where(v == m, cand, _IMAX), axis=1, keepdims=True)
    return (logits, draw)


# speedup vs baseline: 1.0415x; 1.0061x over previous
import functools

import jax
import jax.numpy as jnp
import numpy as np
from jax import lax
from jax.experimental import pallas as pl
from jax.experimental.pallas import tpu as pltpu
from jax.experimental.pallas import tpu_sc as plsc

DIM = 128
NUM_ACTION = 100000
BATCH = 1024
CB = 1024
NBLK = (NUM_ACTION + CB - 1) // CB

TC_BLKS = 74
SC_COL0 = TC_BLKS * CB
SC_COLS = NUM_ACTION - SC_COL0
SC_NW = 32
SC_ROWS = BATCH // SC_NW
SC_BCMAX = 6080
SC_BLOCKS = ((0, 6048), (6048, 6048), (12096, 6048), (18144, 6080))

_TINY = np.float32(np.finfo(np.float32).tiny)
_KS1 = np.uint32(42)
_KS2 = np.uint32(0x1BD11BDA) ^ np.uint32(42)
_ROT_A = (13, 15, 26, 6)
_ROT_B = (17, 29, 16, 24)
_LN2 = np.float32(0.6931471805599453)
_SQRT2 = np.float32(1.4142135623730951)
_IMAX = np.int32(2**31 - 1)


def _rotl(x, d):
    return (x << np.uint32(d)) | (x >> np.uint32(32 - d))


def _rounds(x0, x1, rots):
    for r in rots:
        x0 = x0 + x1
        x1 = _rotl(x1, r)
        x1 = x0 ^ x1
    return x0, x1


def _uniform_bits(flat_idx_u32):
    x0 = jnp.zeros_like(flat_idx_u32)
    x1 = flat_idx_u32 + _KS1
    x0, x1 = _rounds(x0, x1, _ROT_A)
    x0 = x0 + _KS1
    x1 = x1 + (_KS2 + np.uint32(1))
    x0, x1 = _rounds(x0, x1, _ROT_B)
    x0 = x0 + _KS2
    x1 = x1 + np.uint32(2)
    x0, x1 = _rounds(x0, x1, _ROT_A)
    x1 = x1 + (_KS1 + np.uint32(3))
    x0, x1 = _rounds(x0, x1, _ROT_B)
    x0 = x0 + _KS1
    x1 = x1 + (_KS2 + np.uint32(4))
    x0, x1 = _rounds(x0, x1, _ROT_A)
    x0 = x0 + _KS2
    x1 = x1 + np.uint32(5)
    bits = x0 ^ x1
    float_bits = (bits >> np.uint32(9)) | np.uint32(0x3F800000)
    floats = jax.lax.bitcast_convert_type(float_bits, jnp.float32) - np.float32(1.0)
    return jnp.maximum(floats, _TINY)


def _gumbel(flat_idx_u32):
    return -jnp.log(-jnp.log(_uniform_bits(flat_idx_u32)))


def _tc_body(feature_ref, w1t_ref, b1_ref, w2_ref, b2_ref,
             logits_ref, draw_ref, drawv_ref, h_sc, bestv_sc, besti_sc):
    j = pl.program_id(0)

    @pl.when(j == 0)
    def _init():
        h = jnp.dot(feature_ref[...], w1t_ref[...],
                    preferred_element_type=jnp.float32)
        h_sc[...] = jnp.maximum(h + b1_ref[...], 0.0)
        bestv_sc[...] = jnp.full((BATCH, 1), -jnp.inf, jnp.float32)
        besti_sc[...] = jnp.zeros((BATCH, 1), jnp.int32)

    h = h_sc[...]
    logits = jax.lax.dot_general(h, w2_ref[...], (((1,), (1,)), ((), ())),
                                 preferred_element_type=jnp.float32)
    logits = logits + b2_ref[...]
    logits_ref[...] = logits

    @pl.when(j < TC_BLKS)
    def _rng():
        col = j * CB + jax.lax.broadcasted_iota(jnp.int32, (BATCH, CB), 1)
        row = jax.lax.broadcasted_iota(jnp.int32, (BATCH, CB), 0)
        flat = (row * NUM_ACTION + col).astype(jnp.uint32)
        v = logits + _gumbel(flat)
        m = jnp.max(v, axis=1, keepdims=True)
        idx = jnp.min(jnp.where(v == m, col, _IMAX), axis=1, keepdims=True)
        better = m > bestv_sc[...]
        bestv_sc[...] = jnp.where(better, m, bestv_sc[...])
        besti_sc[...] = jnp.where(better, idx, besti_sc[...])

    @pl.when(j == NBLK - 1)
    def _fin():
        draw_ref[...] = besti_sc[...]
        drawv_ref[...] = bestv_sc[...]


def _log_poly(x):
    bits = jax.lax.bitcast_convert_type(x, jnp.int32)
    e = ((bits + np.int32(0x4AFB0D)) >> np.int32(23)) - np.int32(127)
    m = jax.lax.bitcast_convert_type(bits - (e << np.int32(23)), jnp.float32)
    z = m - np.float32(1.0)
    w = z / (z + np.float32(2.0))
    w2 = w * w
    p = w2 * np.float32(2.0 / 9.0) + np.float32(2.0 / 7.0)
    p = p * w2 + np.float32(2.0 / 5.0)
    p = p * w2 + np.float32(2.0 / 3.0)
    p = p * w2 + np.float32(2.0)
    return e.astype(jnp.float32) * _LN2 + w * p


def _gumbel_sc(flat_idx_u32):
    u = _uniform_bits(flat_idx_u32)
    t = -_log_poly(u)
    return -_log_poly(t)


def _sc_kernel_body(featf, w1f, b1p, w2tf, b2f, outv, outi,
                    feat_v, w1_v, b1_v, w2_v, b2_v, h_v, bv_v, bi_v):
    wid = lax.axis_index("s") * np.int32(2) + lax.axis_index("c")
    r0 = wid * SC_ROWS
    pltpu.sync_copy(featf.at[pl.ds(r0 * DIM, SC_ROWS * DIM)], feat_v)
    pltpu.sync_copy(w1f, w1_v)
    pltpu.sync_copy(b1p, b1_v)

    iota16 = lax.iota(jnp.int32, 16)
    neg_inf = jnp.full((16,), -jnp.inf, jnp.float32)
    zeros_i = jnp.zeros((16,), jnp.int32)
    b1vec = b1_v[...]

    def compute_h(r, carry):
        for k in range(8):
            acc = jnp.zeros((16,), jnp.float32)
            for dd in range(DIM // 16):
                f = feat_v[pl.ds(r * DIM + dd * 16, 16)]
                w = w1_v[pl.ds(k * DIM + dd * 16, 16)]
                acc = acc + f * w
            for sh in (8, 4, 2, 1):
                acc = acc + acc.at[iota16 ^ sh].get(mode="promise_in_bounds")
            hk = jnp.maximum(acc + b1vec[k], np.float32(0.0))
            h_v[pl.ds((r * 8 + k) * 16, 16)] = hk
        bv_v[pl.ds(r * 16, 16)] = neg_inf
        bi_v[pl.ds(r * 16, 16)] = zeros_i
        return carry

    lax.fori_loop(0, SC_ROWS, compute_h, 0)

    for off, bc in SC_BLOCKS:
        for k in range(8):
            pltpu.sync_copy(
                w2tf.at[pl.ds(k * SC_COLS + off, bc)],
                w2_v.at[pl.ds(k * SC_BCMAX, bc)])
        pltpu.sync_copy(b2f.at[pl.ds(off, bc)], b2_v.at[pl.ds(0, bc)])
        colbase = SC_COL0 + off

        def rowpair_loop(rp, carry2):
            rA = rp * np.int32(2)
            rB = rA + np.int32(1)
            hA = [h_v[pl.ds((rA * 8 + k) * 16, 16)] for k in range(8)]
            hB = [h_v[pl.ds((rB * 8 + k) * 16, 16)] for k in range(8)]
            rowoffA = (r0 + rA) * NUM_ACTION
            rowoffB = (r0 + rB) * NUM_ACTION

            def chunk_loop(i, st):
                bvA, biA, bvB, biB = st
                for cc in range(2):
                    c16 = (i * 2 + cc) * 16
                    w2c = [w2_v[pl.ds(k * SC_BCMAX + c16, 16)] for k in range(8)]
                    b2c = b2_v[pl.ds(c16, 16)]
                    colv = colbase + c16 + iota16
                    for which in (0, 1):
                        hh = hA if which == 0 else hB
                        rowoff = rowoffA if which == 0 else rowoffB
                        l = b2c
                        for k in range(8):
                            l = l + hh[k] * w2c[k]
                        flat = (rowoff + colv).astype(jnp.uint32)
                        v = l + _gumbel_sc(flat)
                        if which == 0:
                            better = v > bvA
                            bvA = jnp.where(better, v, bvA)
                            biA = jnp.where(better, colv, biA)
                        else:
                            better = v > bvB
                            bvB = jnp.where(better, v, bvB)
                            biB = jnp.where(better, colv, biB)
                return bvA, biA, bvB, biB

            st0 = (bv_v[pl.ds(rA * 16, 16)], bi_v[pl.ds(rA * 16, 16)],
                   bv_v[pl.ds(rB * 16, 16)], bi_v[pl.ds(rB * 16, 16)])
            bvA, biA, bvB, biB = lax.fori_loop(0, bc // 32, chunk_loop, st0)
            bv_v[pl.ds(rA * 16, 16)] = bvA
            bi_v[pl.ds(rA * 16, 16)] = biA
            bv_v[pl.ds(rB * 16, 16)] = bvB
            bi_v[pl.ds(rB * 16, 16)] = biB
            return carry2

        lax.fori_loop(0, SC_ROWS // 2, rowpair_loop, 0)

    pltpu.sync_copy(bv_v, outv.at[pl.ds(r0 * 16, SC_ROWS * 16)])
    pltpu.sync_copy(bi_v, outi.at[pl.ds(r0 * 16, SC_ROWS * 16)])


def _make_sc_call():
    return functools.partial(
        pl.kernel,
        mesh=plsc.VectorSubcoreMesh(core_axis_name="c", subcore_axis_name="s"),
        out_type=[jax.ShapeDtypeStruct((BATCH * 16,), jnp.float32),
                  jax.ShapeDtypeStruct((BATCH * 16,), jnp.int32)],
        scratch_types=[
            pltpu.VMEM((SC_ROWS * DIM,), jnp.float32),
            pltpu.VMEM((8 * DIM,), jnp.float32),
            pltpu.VMEM((16,), jnp.float32),
            pltpu.VMEM((8 * SC_BCMAX,), jnp.float32),
            pltpu.VMEM((SC_BCMAX,), jnp.float32),
            pltpu.VMEM((SC_ROWS * 8 * 16,), jnp.float32),
            pltpu.VMEM((SC_ROWS * 16,), jnp.float32),
            pltpu.VMEM((SC_ROWS * 16,), jnp.int32),
        ],
    )(_sc_kernel_body)


@jax.jit
def kernel(feature, W1, b1, W2, b2):
    w1t = W1.T
    b1r = b1.reshape(1, 8)
    b2r = b2.reshape(1, NUM_ACTION)
    logits, tc_i, tc_v = pl.pallas_call(
        _tc_body,
        grid=(NBLK,),
        in_specs=[
            pl.BlockSpec((BATCH, DIM), lambda j: (0, 0)),
            pl.BlockSpec((DIM, 8), lambda j: (0, 0)),
            pl.BlockSpec((1, 8), lambda j: (0, 0)),
            pl.BlockSpec((CB, 8), lambda j: (j, 0)),
            pl.BlockSpec((1, CB), lambda j: (0, j)),
        ],
        out_specs=[
            pl.BlockSpec((BATCH, CB), lambda j: (0, j)),
            pl.BlockSpec((BATCH, 1), lambda j: (0, 0)),
            pl.BlockSpec((BATCH, 1), lambda j: (0, 0)),
        ],
        out_shape=[
            jax.ShapeDtypeStruct((BATCH, NUM_ACTION), jnp.float32),
            jax.ShapeDtypeStruct((BATCH, 1), jnp.int32),
            jax.ShapeDtypeStruct((BATCH, 1), jnp.float32),
        ],
        scratch_shapes=[
            pltpu.VMEM((BATCH, 8), jnp.float32),
            pltpu.VMEM((BATCH, 1), jnp.float32),
            pltpu.VMEM((BATCH, 1), jnp.int32),
        ],
    )(feature, w1t, b1r, W2, b2r)

    w2t_sc = W2.T[:, SC_COL0:]
    sc_v, sc_i = _make_sc_call()(feature.reshape(-1), W1.reshape(-1),
                                 jnp.pad(b1, (0, 8)), w2t_sc.reshape(-1),
                                 b2[SC_COL0:])

    del tc_v
    cand = jnp.concatenate([tc_i, sc_i.reshape(BATCH, 16)], axis=1)
    l_cand = jnp.take_along_axis(logits, cand, axis=1)
    rows = jnp.arange(BATCH, dtype=jnp.int32)[:, None]
    flat = (rows * NUM_ACTION + cand).astype(jnp.uint32)
    v = l_cand + _gumbel(flat)
    m = jnp.max(v, axis=1, keepdims=True)
    draw = jnp.min(jnp.where(v == m, cand, _IMAX), axis=1, keepdims=True)
    return (logits, draw)
